# baseline (speedup 1.0000x reference)
import functools

import jax
import jax.numpy as jnp
from jax import lax
from jax.experimental import pallas as pl
from jax.experimental.pallas import tpu as pltpu
from jax.experimental.pallas import tpu_sc as plsc


def _sc_dot_lookup(B, D):
    info = plsc.get_sparse_core_info()
    NC, NS, L = info.num_cores, info.num_subcores, info.num_lanes
    NW = NC * NS
    assert B % NW == 0
    b_per_w = B // NW
    n_chunks = 4
    chunk = b_per_w // n_chunks
    D2 = 2 * D

    mesh = plsc.VectorSubcoreMesh(core_axis_name="c", subcore_axis_name="s")

    @functools.partial(
        pl.kernel,
        mesh=mesh,
        out_type=jax.ShapeDtypeStruct((B,), jnp.float32),
        compiler_params=pltpu.CompilerParams(needs_layout_passes=False),
        scratch_types=[
            pltpu.VMEM((n_chunks, chunk), jnp.int32),
            pltpu.VMEM((n_chunks, chunk), jnp.int32),
            pltpu.VMEM((n_chunks, chunk), jnp.int32),
            pltpu.VMEM((n_chunks, chunk), jnp.int32),
            pltpu.VMEM((chunk, D2), jnp.float32),
            pltpu.VMEM((chunk, D2), jnp.float32),
            pltpu.VMEM((chunk, D2), jnp.float32),
            pltpu.VMEM((chunk, D2), jnp.float32),
            pltpu.VMEM((b_per_w,), jnp.float32),
            pltpu.SemaphoreType.DMA,
            pltpu.SemaphoreType.DMA,
            pltpu.SemaphoreType.DMA,
            pltpu.SemaphoreType.DMA,
        ],
    )
    def k(tgt_hbm, ctx_hbm, ttab_hbm, ctab_hbm, out_hbm,
          idx_t, idx_c, sidx_t, sidx_c, rt0, rt1, rc0, rc1, out_v,
          sem_t0, sem_t1, sem_c0, sem_c1):
        wid = lax.axis_index("s") * NC + lax.axis_index("c")
        base = wid * b_per_w
        rows_t = (rt0, rt1)
        rows_c = (rc0, rc1)
        sems_t = (sem_t0, sem_t1)
        sems_c = (sem_c0, sem_c1)

        for j in range(n_chunks):
            pltpu.sync_copy(tgt_hbm.at[pl.ds(base + j * chunk, chunk)], idx_t.at[j])
            pltpu.sync_copy(ctx_hbm.at[pl.ds(base + j * chunk, chunk)], idx_c.at[j])

        for j in range(n_chunks):
            for g in range(chunk // L):
                sl = pl.ds(g * L, L)
                sidx_t[j, sl] = lax.shift_right_logical(idx_t[j, sl], 1)
                sidx_c[j, sl] = lax.shift_right_logical(idx_c[j, sl], 1)

        def fire(j):
            b = j % 2
            ct = pltpu.async_copy(ttab_hbm.at[sidx_t.at[j]], rows_t[b], sems_t[b])
            cc = pltpu.async_copy(ctab_hbm.at[sidx_c.at[j]], rows_c[b], sems_c[b])
            return ct, cc

        lanes = lax.iota(jnp.int32, L)
        inflight = [fire(0), fire(1)]

        for j in range(n_chunks):
            b = j % 2
            ct, cc = inflight[b]
            ct.wait()
            cc.wait()

            def body(g, carry, j=j, b=b):
                sl = pl.ds(g * L, L)
                row = g * L + lanes
                half_t = jnp.bitwise_and(idx_t[j, sl], 1) * D
                half_c = jnp.bitwise_and(idx_c[j, sl], 1) * D
                acc = jnp.zeros((L,), jnp.float32)
                for d in range(D):
                    diag = jnp.bitwise_and(lanes + d, D - 1)
                    tg = plsc.load_gather(rows_t[b], [row, half_t + diag])
                    cg = plsc.load_gather(rows_c[b], [row, half_c + diag])
                    acc = acc + tg * cg
                out_v[pl.ds(j * chunk + g * L, L)] = acc
                return carry

            lax.fori_loop(0, chunk // L, body, 0)

            if j + 2 < n_chunks:
                inflight[b] = fire(j + 2)

        pltpu.sync_copy(out_v, out_hbm.at[pl.ds(base, b_per_w)])

    return k


def kernel(target, context, target_table, context_table):
    B = target.shape[0]
    V, D = target_table.shape
    k = _sc_dot_lookup(B, D)
    return k(target.astype(jnp.int32), context.astype(jnp.int32),
             target_table.reshape(V // 2, 2 * D),
             context_table.reshape(V // 2, 2 * D))
